# baseline (speedup 1.0000x reference)
import functools

import jax
import jax.numpy as jnp
from jax import lax
from jax.experimental import pallas as pl
from jax.experimental.pallas import tpu as pltpu
from jax.experimental.pallas import tpu_sc as plsc

E = 8
NG = 2
GS = E // NG
D = 1024
FF = 512
RSF = 2.5
T = 2048

TB = 256
NBLK = T // TB
NCORE = 2
NSUB = 16
CHUNK = 64
THALF = T // NCORE


def _dot_t(a, b):
    return jax.lax.dot_general(a, b, (((1,), (1,)), ((), ())),
                               preferred_element_type=jnp.float32)


def _sc_mesh():
    return plsc.VectorSubcoreMesh(core_axis_name="c", subcore_axis_name="s",
                                  num_cores=NCORE, num_subcores=NSUB)


def _logits_body(gw_ref, x_ref, o_ref):
    lt = jax.lax.dot_general(
        gw_ref[...], x_ref[...], (((1,), (1,)), ((), ())),
        preferred_element_type=jnp.float32)
    nw = T // CHUNK
    o_ref[...] = jnp.transpose(lt.reshape(E, nw, CHUNK), (1, 0, 2))


def _logits(gw, x):
    nw = T // CHUNK
    return pl.pallas_call(
        _logits_body,
        out_shape=jax.ShapeDtypeStruct((nw, E, CHUNK), jnp.float32),
    )(gw, x)


def _route_body(lt_hbm, b16_hbm, combt_hbm, lt_v, b_v, comb_v):
    cid = lax.axis_index("c")
    sid = lax.axis_index("s")
    wid = cid * NSUB + sid
    tbase = wid * CHUNK

    pltpu.sync_copy(lt_hbm.at[pl.ds(wid * (E * CHUNK), E * CHUNK)], lt_v)
    pltpu.sync_copy(b16_hbm, b_v)

    negbig = jnp.zeros((16,), jnp.float32) - 1e30
    zero_f = jnp.zeros((16,), jnp.float32)
    zero_i = jnp.zeros((16,), jnp.int32)
    nine_v = jnp.zeros((16,), jnp.int32) + (E + 1)
    e_vs = [jnp.zeros((16,), jnp.int32) + e for e in range(E)]
    one_i = jnp.zeros((16,), jnp.int32) + 1
    two_i = one_i + one_i

    for j in range(CHUNK // 16):
        s = [1.0 / (1.0 + jnp.exp(-lt_v[pl.ds(e * CHUNK + j * 16, 16)]))
             for e in range(E)]
        sc = [s[e] + b_v[pl.ds(e * 16, 16)] for e in range(E)]

        def top2sum(grp):
            m1 = jnp.maximum(jnp.maximum(grp[0], grp[1]),
                             jnp.maximum(grp[2], grp[3]))
            eqs = [g == m1 for g in grp]
            neq = zero_i
            for eq in eqs:
                neq = neq + jnp.where(eq, one_i, zero_i)
            m2c = [jnp.where(eq, negbig, g) for eq, g in zip(eqs, grp)]
            m2 = jnp.maximum(jnp.maximum(m2c[0], m2c[1]),
                             jnp.maximum(m2c[2], m2c[3]))
            m2 = jnp.where(neq >= two_i, m1, m2)
            return m1 + m2

        g0 = top2sum(sc[0:GS])
        g1 = top2sum(sc[GS:E])
        sel0 = g0 >= g1
        sel1 = g1 > g0
        v = [jnp.where(sel0 if e < GS else sel1, sc[e], zero_f)
             for e in range(E)]
        m1v = v[0]
        for e in range(1, E):
            m1v = jnp.maximum(m1v, v[e])
        idx1 = nine_v
        for e in range(E):
            idx1 = jnp.minimum(idx1, jnp.where(v[e] == m1v, e_vs[e], nine_v))
        w1 = zero_f
        for e in range(E):
            w1 = w1 + jnp.where(idx1 == e_vs[e], s[e], zero_f)
        v2 = [jnp.where(idx1 == e_vs[e], negbig, v[e]) for e in range(E)]
        m2v = v2[0]
        for e in range(1, E):
            m2v = jnp.maximum(m2v, v2[e])
        idx2 = nine_v
        for e in range(E):
            idx2 = jnp.minimum(idx2, jnp.where(v2[e] == m2v, e_vs[e],
                                               nine_v))
        w2 = zero_f
        for e in range(E):
            w2 = w2 + jnp.where(idx2 == e_vs[e], s[e], zero_f)
        scale = jnp.float32(RSF) / (w1 + w2 + jnp.float32(1e-20))
        w1f = w1 * scale
        w2f = w2 * scale
        for e in range(E):
            ce = (jnp.where(idx1 == e_vs[e], w1f, zero_f)
                  + jnp.where(idx2 == e_vs[e], w2f, zero_f))
            comb_v[pl.ds(e * CHUNK + j * 16, 16)] = ce

    for e in range(E):
        pltpu.sync_copy(comb_v.at[pl.ds(e * CHUNK, CHUNK)],
                        combt_hbm.at[pl.ds(e * T + tbase, CHUNK)])


def _route(logits_flat, b16):
    run = functools.partial(
        pl.kernel,
        out_type=[jax.ShapeDtypeStruct((E * T,), jnp.float32)],
        mesh=_sc_mesh(),
        scratch_types=[
            pltpu.VMEM((E * CHUNK,), jnp.float32),
            pltpu.VMEM((E * 16,), jnp.float32),
            pltpu.VMEM((E * CHUNK,), jnp.float32),
        ],
    )(_route_body)
    (combt,) = run(logits_flat, b16)
    return combt


def _shared_body(x_ref, sg_ref, su_ref, sd_ref, o_ref):
    xb = x_ref[...].astype(jnp.bfloat16)
    hg = _dot_t(xb, sg_ref[...].astype(jnp.bfloat16))
    hu = _dot_t(xb, su_ref[...].astype(jnp.bfloat16))
    h = (jax.nn.silu(hg) * hu).astype(jnp.bfloat16)
    o_ref[...] = _dot_t(h, sd_ref[...].astype(jnp.bfloat16))


def _shared(x, sg, su, sd):
    return pl.pallas_call(
        _shared_body,
        grid=(NBLK,),
        in_specs=[
            pl.BlockSpec((TB, D), lambda i: (i, 0)),
            pl.BlockSpec((FF, D), lambda i: (0, 0)),
            pl.BlockSpec((FF, D), lambda i: (0, 0)),
            pl.BlockSpec((D, FF), lambda i: (0, 0)),
        ],
        out_specs=pl.BlockSpec((TB, D), lambda i: (i, 0)),
        out_shape=jax.ShapeDtypeStruct((T, D), jnp.float32),
        compiler_params=pltpu.CompilerParams(
            dimension_semantics=("arbitrary",)),
    )(x, sg, su, sd)


def _moe_body(x_ref, combt_ref, sh_ref, eg_ref, eu_ref, ed_ref,
              o_ref, acc_ref, xb_ref, comb_ref):
    e = pl.program_id(0)
    i = pl.program_id(1)
    rows = pl.ds(i * TB, TB)

    @pl.when(e == 0)
    def _init():
        @pl.when(i == 0)
        def _tr():
            comb_ref[...] = combt_ref[...].T

        xb_ref[rows, :] = x_ref[rows, :].astype(jnp.bfloat16)
        acc_ref[rows, :] = sh_ref[rows, :]

    xb = xb_ref[rows, :]
    hg = _dot_t(xb, eg_ref[0].astype(jnp.bfloat16))
    hu = _dot_t(xb, eu_ref[0].astype(jnp.bfloat16))
    h = (jax.nn.silu(hg) * hu).astype(jnp.bfloat16)
    eo = _dot_t(h, ed_ref[0].astype(jnp.bfloat16))
    cols = jax.lax.broadcasted_iota(jnp.int32, (TB, E), 1)
    ce = jnp.sum(jnp.where(cols == e, comb_ref[rows, :], 0.0), axis=1,
                 keepdims=True)
    acc_ref[rows, :] = acc_ref[rows, :] + eo * ce

    @pl.when(e == E - 1)
    def _fin():
        o_ref[...] = acc_ref[rows, :]


def _moe_dense(x, combt, sh, eg, eu, ed):
    grid = (E, NBLK)
    return pl.pallas_call(
        _moe_body,
        grid=grid,
        in_specs=[
            pl.BlockSpec((T, D), lambda e, i: (0, 0)),
            pl.BlockSpec((E, T), lambda e, i: (0, 0)),
            pl.BlockSpec((T, D), lambda e, i: (0, 0)),
            pl.BlockSpec((1, FF, D), lambda e, i: (e, 0, 0)),
            pl.BlockSpec((1, FF, D), lambda e, i: (e, 0, 0)),
            pl.BlockSpec((1, D, FF), lambda e, i: (e, 0, 0)),
        ],
        out_specs=pl.BlockSpec(
            (TB, D), lambda e, i: (jnp.where(e == E - 1, i, 0), 0)),
        out_shape=jax.ShapeDtypeStruct((T, D), jnp.float32),
        scratch_shapes=[
            pltpu.VMEM((T, D), jnp.float32),
            pltpu.VMEM((T, D), jnp.bfloat16),
            pltpu.VMEM((T, E), jnp.float32),
        ],
        compiler_params=pltpu.CompilerParams(
            dimension_semantics=("arbitrary", "arbitrary"),
        ),
    )(x, combt, sh, eg, eu, ed)


@jax.jit
def _moe(x, gate_weight, bias, eg, eu, ed, sg, su, sd):
    logits = _logits(gate_weight, x)
    b16 = jnp.broadcast_to(bias.reshape(E, 1), (E, 16)).reshape(E * 16)
    combt = _route(logits.reshape(-1), b16)
    sh = _shared(x, sg, su, sd)
    return _moe_dense(x, combt.reshape(E, T), sh, eg, eu, ed)


def kernel(hidden_states, gate_weight, e_score_correction_bias,
           expert_gate_w, expert_up_w, expert_down_w,
           shared_gate_w, shared_up_w, shared_down_w):
    orig_shape = hidden_states.shape
    x = hidden_states.reshape(-1, D).astype(jnp.float32)
    out = _moe(x, gate_weight, e_score_correction_bias,
               expert_gate_w, expert_up_w, expert_down_w,
               shared_gate_w, shared_up_w, shared_down_w)
    return out.reshape(orig_shape)


# speedup vs baseline: 1.2068x; 1.2068x over previous
import functools

import jax
import jax.numpy as jnp
from jax import lax
from jax.experimental import pallas as pl
from jax.experimental.pallas import tpu as pltpu
from jax.experimental.pallas import tpu_sc as plsc

E = 8
NG = 2
GS = E // NG
D = 1024
FF = 512
RSF = 2.5
T = 2048

TB = 512
NBLK = T // TB
NCORE = 2
NSUB = 16
CHUNK = 64
THALF = T // NCORE


def _dot_t(a, b):
    return jax.lax.dot_general(a, b, (((1,), (1,)), ((), ())),
                               preferred_element_type=jnp.float32)


def _sc_mesh():
    return plsc.VectorSubcoreMesh(core_axis_name="c", subcore_axis_name="s",
                                  num_cores=NCORE, num_subcores=NSUB)


def _logits_body(gw_ref, x_ref, o_ref):
    lt = jax.lax.dot_general(
        gw_ref[...], x_ref[...], (((1,), (1,)), ((), ())),
        preferred_element_type=jnp.float32)
    nw = T // CHUNK
    o_ref[...] = jnp.transpose(lt.reshape(E, nw, CHUNK), (1, 0, 2))


def _logits(gw, x):
    nw = T // CHUNK
    return pl.pallas_call(
        _logits_body,
        out_shape=jax.ShapeDtypeStruct((nw, E, CHUNK), jnp.float32),
    )(gw, x)


def _route_body(lt_hbm, b16_hbm, combt_hbm, lt_v, b_v, comb_v):
    cid = lax.axis_index("c")
    sid = lax.axis_index("s")
    wid = cid * NSUB + sid
    tbase = wid * CHUNK

    pltpu.sync_copy(lt_hbm.at[pl.ds(wid * (E * CHUNK), E * CHUNK)], lt_v)
    pltpu.sync_copy(b16_hbm, b_v)

    negbig = jnp.zeros((16,), jnp.float32) - 1e30
    zero_f = jnp.zeros((16,), jnp.float32)
    zero_i = jnp.zeros((16,), jnp.int32)
    nine_v = jnp.zeros((16,), jnp.int32) + (E + 1)
    e_vs = [jnp.zeros((16,), jnp.int32) + e for e in range(E)]
    one_i = jnp.zeros((16,), jnp.int32) + 1
    two_i = one_i + one_i

    for j in range(CHUNK // 16):
        s = [1.0 / (1.0 + jnp.exp(-lt_v[pl.ds(e * CHUNK + j * 16, 16)]))
             for e in range(E)]
        sc = [s[e] + b_v[pl.ds(e * 16, 16)] for e in range(E)]

        def top2sum(grp):
            m1 = jnp.maximum(jnp.maximum(grp[0], grp[1]),
                             jnp.maximum(grp[2], grp[3]))
            eqs = [g == m1 for g in grp]
            neq = zero_i
            for eq in eqs:
                neq = neq + jnp.where(eq, one_i, zero_i)
            m2c = [jnp.where(eq, negbig, g) for eq, g in zip(eqs, grp)]
            m2 = jnp.maximum(jnp.maximum(m2c[0], m2c[1]),
                             jnp.maximum(m2c[2], m2c[3]))
            m2 = jnp.where(neq >= two_i, m1, m2)
            return m1 + m2

        g0 = top2sum(sc[0:GS])
        g1 = top2sum(sc[GS:E])
        sel0 = g0 >= g1
        sel1 = g1 > g0
        v = [jnp.where(sel0 if e < GS else sel1, sc[e], zero_f)
             for e in range(E)]
        m1v = v[0]
        for e in range(1, E):
            m1v = jnp.maximum(m1v, v[e])
        idx1 = nine_v
        for e in range(E):
            idx1 = jnp.minimum(idx1, jnp.where(v[e] == m1v, e_vs[e], nine_v))
        w1 = zero_f
        for e in range(E):
            w1 = w1 + jnp.where(idx1 == e_vs[e], s[e], zero_f)
        v2 = [jnp.where(idx1 == e_vs[e], negbig, v[e]) for e in range(E)]
        m2v = v2[0]
        for e in range(1, E):
            m2v = jnp.maximum(m2v, v2[e])
        idx2 = nine_v
        for e in range(E):
            idx2 = jnp.minimum(idx2, jnp.where(v2[e] == m2v, e_vs[e],
                                               nine_v))
        w2 = zero_f
        for e in range(E):
            w2 = w2 + jnp.where(idx2 == e_vs[e], s[e], zero_f)
        scale = jnp.float32(RSF) / (w1 + w2 + jnp.float32(1e-20))
        w1f = w1 * scale
        w2f = w2 * scale
        for e in range(E):
            ce = (jnp.where(idx1 == e_vs[e], w1f, zero_f)
                  + jnp.where(idx2 == e_vs[e], w2f, zero_f))
            comb_v[pl.ds(e * CHUNK + j * 16, 16)] = ce

    for e in range(E):
        pltpu.sync_copy(comb_v.at[pl.ds(e * CHUNK, CHUNK)],
                        combt_hbm.at[pl.ds(e * T + tbase, CHUNK)])


def _route(logits_flat, b16):
    run = functools.partial(
        pl.kernel,
        out_type=[jax.ShapeDtypeStruct((E * T,), jnp.float32)],
        mesh=_sc_mesh(),
        scratch_types=[
            pltpu.VMEM((E * CHUNK,), jnp.float32),
            pltpu.VMEM((E * 16,), jnp.float32),
            pltpu.VMEM((E * CHUNK,), jnp.float32),
        ],
    )(_route_body)
    (combt,) = run(logits_flat, b16)
    return combt


def _moe_body(x_ref, combt_ref, eg_ref, eu_ref, ed_ref,
              sg_ref, su_ref, sd_ref, o_ref, acc_ref, xb_ref, comb_ref):
    e = pl.program_id(0)
    i = pl.program_id(1)
    rows = pl.ds(i * TB, TB)

    @pl.when(e == 0)
    def _init():
        @pl.when(i == 0)
        def _tr():
            comb_ref[...] = combt_ref[...].T

        xb = x_ref[rows, :].astype(jnp.bfloat16)
        xb_ref[rows, :] = xb
        hg = _dot_t(xb, sg_ref[...].astype(jnp.bfloat16))
        hu = _dot_t(xb, su_ref[...].astype(jnp.bfloat16))
        h = (jax.nn.silu(hg) * hu).astype(jnp.bfloat16)
        acc_ref[rows, :] = _dot_t(h, sd_ref[...].astype(jnp.bfloat16))

    xb = xb_ref[rows, :]
    hg = _dot_t(xb, eg_ref[0].astype(jnp.bfloat16))
    hu = _dot_t(xb, eu_ref[0].astype(jnp.bfloat16))
    h = (jax.nn.silu(hg) * hu).astype(jnp.bfloat16)
    eo = _dot_t(h, ed_ref[0].astype(jnp.bfloat16))
    cols = jax.lax.broadcasted_iota(jnp.int32, (TB, E), 1)
    ce = jnp.sum(jnp.where(cols == e, comb_ref[rows, :], 0.0), axis=1,
                 keepdims=True)
    acc_ref[rows, :] = acc_ref[rows, :] + eo * ce

    @pl.when(e == E - 1)
    def _fin():
        o_ref[...] = acc_ref[rows, :]


def _moe_dense(x, combt, eg, eu, ed, sg, su, sd):
    grid = (E, NBLK)
    return pl.pallas_call(
        _moe_body,
        grid=grid,
        in_specs=[
            pl.BlockSpec((T, D), lambda e, i: (0, 0)),
            pl.BlockSpec((E, T), lambda e, i: (0, 0)),
            pl.BlockSpec((1, FF, D), lambda e, i: (e, 0, 0)),
            pl.BlockSpec((1, FF, D), lambda e, i: (e, 0, 0)),
            pl.BlockSpec((1, D, FF), lambda e, i: (e, 0, 0)),
            pl.BlockSpec((FF, D), lambda e, i: (0, 0)),
            pl.BlockSpec((FF, D), lambda e, i: (0, 0)),
            pl.BlockSpec((D, FF), lambda e, i: (0, 0)),
        ],
        out_specs=pl.BlockSpec(
            (TB, D), lambda e, i: (jnp.where(e == E - 1, i, 0), 0)),
        out_shape=jax.ShapeDtypeStruct((T, D), jnp.float32),
        scratch_shapes=[
            pltpu.VMEM((T, D), jnp.float32),
            pltpu.VMEM((T, D), jnp.bfloat16),
            pltpu.VMEM((T, E), jnp.float32),
        ],
        compiler_params=pltpu.CompilerParams(
            dimension_semantics=("arbitrary", "arbitrary"),
        ),
    )(x, combt, eg, eu, ed, sg, su, sd)


@jax.jit
def _moe(x, gate_weight, bias, eg, eu, ed, sg, su, sd):
    logits = _logits(gate_weight, x)
    b16 = jnp.broadcast_to(bias.reshape(E, 1), (E, 16)).reshape(E * 16)
    combt = _route(logits.reshape(-1), b16)
    return _moe_dense(x, combt.reshape(E, T), eg, eu, ed, sg, su, sd)


def kernel(hidden_states, gate_weight, e_score_correction_bias,
           expert_gate_w, expert_up_w, expert_down_w,
           shared_gate_w, shared_up_w, shared_down_w):
    orig_shape = hidden_states.shape
    x = hidden_states.reshape(-1, D).astype(jnp.float32)
    out = _moe(x, gate_weight, e_score_correction_bias,
               expert_gate_w, expert_up_w, expert_down_w,
               shared_gate_w, shared_up_w, shared_down_w)
    return out.reshape(orig_shape)


# speedup vs baseline: 1.3687x; 1.1342x over previous
import functools

import jax
import jax.numpy as jnp
from jax import lax
from jax.experimental import pallas as pl
from jax.experimental.pallas import tpu as pltpu
from jax.experimental.pallas import tpu_sc as plsc

E = 8
NG = 2
GS = E // NG
D = 1024
FF = 512
RSF = 2.5
T = 2048

TB = 1024
NBLK = T // TB
NCORE = 2
NSUB = 16
CHUNK = 64
THALF = T // NCORE


def _dot_t(a, b):
    return jax.lax.dot_general(a, b, (((1,), (1,)), ((), ())),
                               preferred_element_type=jnp.float32)


def _sc_mesh():
    return plsc.VectorSubcoreMesh(core_axis_name="c", subcore_axis_name="s",
                                  num_cores=NCORE, num_subcores=NSUB)


def _logits_body(gw_ref, x_ref, o_ref):
    lt = jax.lax.dot_general(
        gw_ref[...], x_ref[...], (((1,), (1,)), ((), ())),
        preferred_element_type=jnp.float32)
    nw = T // CHUNK
    o_ref[...] = jnp.transpose(lt.reshape(E, nw, CHUNK), (1, 0, 2))


def _logits(gw, x):
    nw = T // CHUNK
    return pl.pallas_call(
        _logits_body,
        out_shape=jax.ShapeDtypeStruct((nw, E, CHUNK), jnp.float32),
    )(gw, x)


def _route_body(lt_hbm, b16_hbm, combt_hbm, lt_v, b_v, comb_v):
    cid = lax.axis_index("c")
    sid = lax.axis_index("s")
    wid = cid * NSUB + sid
    tbase = wid * CHUNK

    pltpu.sync_copy(lt_hbm.at[pl.ds(wid * (E * CHUNK), E * CHUNK)], lt_v)
    pltpu.sync_copy(b16_hbm, b_v)

    negbig = jnp.zeros((16,), jnp.float32) - 1e30
    zero_f = jnp.zeros((16,), jnp.float32)
    zero_i = jnp.zeros((16,), jnp.int32)
    nine_v = jnp.zeros((16,), jnp.int32) + (E + 1)
    e_vs = [jnp.zeros((16,), jnp.int32) + e for e in range(E)]
    one_i = jnp.zeros((16,), jnp.int32) + 1
    two_i = one_i + one_i

    for j in range(CHUNK // 16):
        s = [1.0 / (1.0 + jnp.exp(-lt_v[pl.ds(e * CHUNK + j * 16, 16)]))
             for e in range(E)]
        sc = [s[e] + b_v[pl.ds(e * 16, 16)] for e in range(E)]

        def top2sum(grp):
            m1 = jnp.maximum(jnp.maximum(grp[0], grp[1]),
                             jnp.maximum(grp[2], grp[3]))
            eqs = [g == m1 for g in grp]
            neq = zero_i
            for eq in eqs:
                neq = neq + jnp.where(eq, one_i, zero_i)
            m2c = [jnp.where(eq, negbig, g) for eq, g in zip(eqs, grp)]
            m2 = jnp.maximum(jnp.maximum(m2c[0], m2c[1]),
                             jnp.maximum(m2c[2], m2c[3]))
            m2 = jnp.where(neq >= two_i, m1, m2)
            return m1 + m2

        g0 = top2sum(sc[0:GS])
        g1 = top2sum(sc[GS:E])
        sel0 = g0 >= g1
        sel1 = g1 > g0
        v = [jnp.where(sel0 if e < GS else sel1, sc[e], zero_f)
             for e in range(E)]
        m1v = v[0]
        for e in range(1, E):
            m1v = jnp.maximum(m1v, v[e])
        idx1 = nine_v
        for e in range(E):
            idx1 = jnp.minimum(idx1, jnp.where(v[e] == m1v, e_vs[e], nine_v))
        w1 = zero_f
        for e in range(E):
            w1 = w1 + jnp.where(idx1 == e_vs[e], s[e], zero_f)
        v2 = [jnp.where(idx1 == e_vs[e], negbig, v[e]) for e in range(E)]
        m2v = v2[0]
        for e in range(1, E):
            m2v = jnp.maximum(m2v, v2[e])
        idx2 = nine_v
        for e in range(E):
            idx2 = jnp.minimum(idx2, jnp.where(v2[e] == m2v, e_vs[e],
                                               nine_v))
        w2 = zero_f
        for e in range(E):
            w2 = w2 + jnp.where(idx2 == e_vs[e], s[e], zero_f)
        scale = jnp.float32(RSF) / (w1 + w2 + jnp.float32(1e-20))
        w1f = w1 * scale
        w2f = w2 * scale
        for e in range(E):
            ce = (jnp.where(idx1 == e_vs[e], w1f, zero_f)
                  + jnp.where(idx2 == e_vs[e], w2f, zero_f))
            comb_v[pl.ds(e * CHUNK + j * 16, 16)] = ce

    for e in range(E):
        pltpu.sync_copy(comb_v.at[pl.ds(e * CHUNK, CHUNK)],
                        combt_hbm.at[pl.ds(e * T + tbase, CHUNK)])


def _route(logits_flat, b16):
    run = functools.partial(
        pl.kernel,
        out_type=[jax.ShapeDtypeStruct((E * T,), jnp.float32)],
        mesh=_sc_mesh(),
        scratch_types=[
            pltpu.VMEM((E * CHUNK,), jnp.float32),
            pltpu.VMEM((E * 16,), jnp.float32),
            pltpu.VMEM((E * CHUNK,), jnp.float32),
        ],
    )(_route_body)
    (combt,) = run(logits_flat, b16)
    return combt


def _moe_body(x_ref, combt_ref, eg_ref, eu_ref, ed_ref,
              sg_ref, su_ref, sd_ref, o_ref, acc_ref, xb_ref, comb_ref):
    e = pl.program_id(0)
    i = pl.program_id(1)
    rows = pl.ds(i * TB, TB)

    @pl.when(e == 0)
    def _init():
        @pl.when(i == 0)
        def _tr():
            comb_ref[...] = combt_ref[...].T

        xb = x_ref[rows, :].astype(jnp.bfloat16)
        xb_ref[rows, :] = xb
        hg = _dot_t(xb, sg_ref[...].astype(jnp.bfloat16))
        hu = _dot_t(xb, su_ref[...].astype(jnp.bfloat16))
        h = (jax.nn.silu(hg) * hu).astype(jnp.bfloat16)
        acc_ref[rows, :] = _dot_t(h, sd_ref[...].astype(jnp.bfloat16))

    xb = xb_ref[rows, :]
    hg = _dot_t(xb, eg_ref[0].astype(jnp.bfloat16))
    hu = _dot_t(xb, eu_ref[0].astype(jnp.bfloat16))
    h = (jax.nn.silu(hg) * hu).astype(jnp.bfloat16)
    eo = _dot_t(h, ed_ref[0].astype(jnp.bfloat16))
    cols = jax.lax.broadcasted_iota(jnp.int32, (TB, E), 1)
    ce = jnp.sum(jnp.where(cols == e, comb_ref[rows, :], 0.0), axis=1,
                 keepdims=True)
    acc_ref[rows, :] = acc_ref[rows, :] + eo * ce

    @pl.when(e == E - 1)
    def _fin():
        o_ref[...] = acc_ref[rows, :]


def _moe_dense(x, combt, eg, eu, ed, sg, su, sd):
    grid = (E, NBLK)
    return pl.pallas_call(
        _moe_body,
        grid=grid,
        in_specs=[
            pl.BlockSpec((T, D), lambda e, i: (0, 0)),
            pl.BlockSpec((E, T), lambda e, i: (0, 0)),
            pl.BlockSpec((1, FF, D), lambda e, i: (e, 0, 0)),
            pl.BlockSpec((1, FF, D), lambda e, i: (e, 0, 0)),
            pl.BlockSpec((1, D, FF), lambda e, i: (e, 0, 0)),
            pl.BlockSpec((FF, D), lambda e, i: (0, 0)),
            pl.BlockSpec((FF, D), lambda e, i: (0, 0)),
            pl.BlockSpec((D, FF), lambda e, i: (0, 0)),
        ],
        out_specs=pl.BlockSpec(
            (TB, D), lambda e, i: (jnp.where(e == E - 1, i, 0), 0)),
        out_shape=jax.ShapeDtypeStruct((T, D), jnp.float32),
        scratch_shapes=[
            pltpu.VMEM((T, D), jnp.float32),
            pltpu.VMEM((T, D), jnp.bfloat16),
            pltpu.VMEM((T, E), jnp.float32),
        ],
        compiler_params=pltpu.CompilerParams(
            dimension_semantics=("arbitrary", "arbitrary"),
        ),
    )(x, combt, eg, eu, ed, sg, su, sd)


@jax.jit
def _moe(x, gate_weight, bias, eg, eu, ed, sg, su, sd):
    logits = _logits(gate_weight, x)
    b16 = jnp.broadcast_to(bias.reshape(E, 1), (E, 16)).reshape(E * 16)
    combt = _route(logits.reshape(-1), b16)
    return _moe_dense(x, combt.reshape(E, T), eg, eu, ed, sg, su, sd)


def kernel(hidden_states, gate_weight, e_score_correction_bias,
           expert_gate_w, expert_up_w, expert_down_w,
           shared_gate_w, shared_up_w, shared_down_w):
    orig_shape = hidden_states.shape
    x = hidden_states.reshape(-1, D).astype(jnp.float32)
    out = _moe(x, gate_weight, e_score_correction_bias,
               expert_gate_w, expert_up_w, expert_down_w,
               shared_gate_w, shared_up_w, shared_down_w)
    return out.reshape(orig_shape)
